# baseline (speedup 1.0000x reference)
import dataclasses
import functools

import jax
import jax.numpy as jnp
from jax import lax
from jax.experimental import pallas as pl
from jax.experimental.pallas import tpu as pltpu
from jax.experimental.pallas import tpu_sc as plsc

N = 16384
D = 64
C = 100
C_PAD = 112
NC, NS = 2, 16
NW = NC * NS
ROWS_PER_W = N // NW
GROUPS = 4
GROUP = ROWS_PER_W // GROUPS
DELTA = 100.0


def _sc_segment_sums(emb4, lab2):
  mesh = plsc.VectorSubcoreMesh(
      core_axis_name="c", subcore_axis_name="s", num_cores=NC, num_subcores=NS
  )
  cp = pltpu.CompilerParams()
  if "needs_layout_passes" in pltpu.CompilerParams.__dataclass_fields__:
    cp = dataclasses.replace(cp, needs_layout_passes=False)

  @functools.partial(
      pl.kernel,
      out_type=jax.ShapeDtypeStruct((NW, C_PAD, D), jnp.float32),
      mesh=mesh,
      compiler_params=cp,
      scratch_types=[
          pltpu.VMEM((GROUPS, GROUP, D), jnp.float32),
          pltpu.VMEM((ROWS_PER_W,), jnp.int32),
          pltpu.VMEM((C_PAD, D), jnp.float32),
          pltpu.SemaphoreType.DMA,
      ],
  )
  def seg_kernel(emb_hbm, lab_hbm, out_sum, rows_v, lbl_v, acc_v, sem):
    cid = lax.axis_index("c")
    sid = lax.axis_index("s")
    wid = cid * NS + sid

    rows_cp = pltpu.async_copy(emb_hbm.at[wid], rows_v, sem)
    pltpu.sync_copy(lab_hbm.at[wid], lbl_v)

    zero16 = jnp.zeros((16,), jnp.float32)

    @pl.loop(0, C_PAD)
    def _(r):
      @pl.loop(0, D // 16)
      def _(j):
        acc_v[r, pl.ds(j * 16, 16)] = zero16

    rows_cp.wait()

    col_iota = lax.iota(jnp.int32, 16)
    for g in range(GROUPS):

      @pl.loop(0, GROUP, unroll=2)
      def _(r, g=g):
        lbl = plsc.load_gather(lbl_v, [jnp.full((16,), g * GROUP, jnp.int32) + r])
        for j in range(D // 16):
          v = rows_v[g, r, pl.ds(j * 16, 16)]
          plsc.addupdate_scatter(acc_v, [lbl, col_iota + (j * 16)], v)

    pltpu.sync_copy(acc_v, out_sum.at[wid])

  return seg_kernel(emb4, lab2)


def _tc_sumsq(emb):
  blocks = 8

  def body(x_ref, o_ref):
    @pl.when(pl.program_id(0) == 0)
    def _():
      o_ref[0, 0] = 0.0

    x = x_ref[...]
    o_ref[0, 0] += jnp.sum(x * x)

  return pl.pallas_call(
      body,
      grid=(blocks,),
      in_specs=[pl.BlockSpec((N // blocks, D), lambda i: (i, 0))],
      out_specs=pl.BlockSpec(memory_space=pltpu.SMEM),
      out_shape=jax.ShapeDtypeStruct((1, 1), jnp.float32),
  )(emb)


def _tc_counts(lab2):
  def body(l_ref, o_ref):
    labs = l_ref[...]
    classes = lax.broadcasted_iota(jnp.int32, (C_PAD, 1, 1), 0)
    eq = (labs[None, :, :] == classes).astype(jnp.float32)
    o_ref[...] = jnp.sum(eq, axis=2)

  return pl.pallas_call(
      body,
      out_shape=jax.ShapeDtypeStruct((C_PAD, 128), jnp.float32),
  )(lab2)


def _tc_finish(psum, pcnt, ssq, nc_arr):
  def body(ps_ref, pc_ref, ssq_ref, nc_ref, o_ref):
    sums = ps_ref[0]
    for w in range(1, NW):
      sums = sums + ps_ref[w]
    cnt = jnp.sum(pc_ref[...], axis=1, keepdims=True)
    safe = jnp.maximum(cnt, 1.0)
    mean = sums / safe
    wnorm = jnp.sum(sums * sums, axis=1, keepdims=True) / safe
    intra = ssq_ref[0, 0] - jnp.sum(wnorm)
    pm = mean[:, None, :] - mean[None, :, :]
    d2 = jnp.sum(pm * pm, axis=-1)
    ii = lax.broadcasted_iota(jnp.int32, (C_PAD, 1), 0)
    nonempty = (cnt > 0.0) & (ii < nc_ref[0, 0])
    ri = lax.broadcasted_iota(jnp.int32, (C_PAD, C_PAD), 0)
    ci = lax.broadcasted_iota(jnp.int32, (C_PAD, C_PAD), 1)
    pair_mask = nonempty & nonempty.reshape(1, C_PAD) & (ri != ci)
    inter = jnp.min(jnp.where(pair_mask, d2, jnp.inf))
    loss = intra / N - jnp.minimum(DELTA, inter)
    o_ref[0, 0] = loss

  return pl.pallas_call(
      body,
      in_specs=[
          pl.BlockSpec(memory_space=pltpu.VMEM),
          pl.BlockSpec(memory_space=pltpu.VMEM),
          pl.BlockSpec(memory_space=pltpu.SMEM),
          pl.BlockSpec(memory_space=pltpu.SMEM),
      ],
      out_specs=pl.BlockSpec(memory_space=pltpu.SMEM),
      out_shape=jax.ShapeDtypeStruct((1, 1), jnp.float32),
  )(psum, pcnt, ssq, nc_arr)


def kernel(embeddings, labels, num_classes):
  emb = embeddings.astype(jnp.float32)
  lab = labels.astype(jnp.int32)
  emb4 = emb.reshape(NW, GROUPS, GROUP, D)
  lab2 = lab.reshape(NW, ROWS_PER_W)
  psum = _sc_segment_sums(emb4, lab2)
  ssq = _tc_sumsq(emb)
  pcnt = _tc_counts(lab.reshape(128, 128))
  nc_arr = jnp.asarray(num_classes, jnp.int32).reshape(1, 1)
  loss = _tc_finish(psum, pcnt, ssq, nc_arr)
  return loss.reshape(1)
